# Initial kernel scaffold; baseline (speedup 1.0000x reference)
import jax
import jax.numpy as jnp
from jax.experimental import pallas as pl


def kernel(x, table, ln_weight, ln_bias):
    raise NotImplementedError("write your pallas kernel here")


# speedup vs baseline: 1.2628x; 1.2628x over previous
import functools

import jax
import jax.numpy as jnp
from jax import lax
from jax.experimental import pallas as pl
from jax.experimental.pallas import tpu as pltpu
from jax.experimental.pallas import tpu_sc as plsc

D = 64
LANES = 16
CHUNK = 128
EPS = 1e-5


def _rsqrt(x):
    i = lax.bitcast_convert_type(x, jnp.int32)
    i = jnp.int32(0x5F3759DF) - (i >> 1)
    y = lax.bitcast_convert_type(i, jnp.float32)
    for _ in range(3):
        y = y * (1.5 - (0.5 * x) * y * y)
    return y


@functools.cache
def _make_sc_call(n_rows):
    info = plsc.get_sparse_core_info()
    num_cores = info.num_cores
    nw = num_cores * info.num_subcores
    per_w = n_rows // nw
    n_chunks = per_w // CHUNK
    assert per_w * nw == n_rows and n_chunks * CHUNK == per_w
    mesh = plsc.VectorSubcoreMesh(core_axis_name="c", subcore_axis_name="s")

    @functools.partial(
        pl.kernel,
        mesh=mesh,
        compiler_params=pltpu.CompilerParams(
            needs_layout_passes=False, use_tc_tiling_on_sc=False),
        out_type=jax.ShapeDtypeStruct((n_rows, D), jnp.float32),
        scratch_types=[
            pltpu.VMEM((CHUNK,), jnp.int32),
            pltpu.VMEM((CHUNK, D), jnp.float32),
            pltpu.VMEM((D,), jnp.float32),
            pltpu.VMEM((D,), jnp.float32),
            pltpu.SemaphoreType.DMA,
        ],
    )
    def body(idx_hbm, table_hbm, w_hbm, b_hbm, out_hbm,
             idx_v, rows_v, w_v, b_v, sem):
        wid = lax.axis_index("s") * num_cores + lax.axis_index("c")
        base = wid * per_w
        pltpu.sync_copy(w_hbm, w_v)
        pltpu.sync_copy(b_hbm, b_v)

        def chunk_body(ci, carry):
            off = base + ci * CHUNK
            pltpu.sync_copy(idx_hbm.at[pl.ds(off, CHUNK)], idx_v)
            pltpu.async_copy(table_hbm.at[idx_v], rows_v, sem).wait()
            wb = [(w_v[pl.ds(j * LANES, LANES)], b_v[pl.ds(j * LANES, LANES)])
                  for j in range(D // LANES)]
            for r in range(CHUNK):
                vs = [rows_v[r, pl.ds(j * LANES, LANES)]
                      for j in range(D // LANES)]
                s = vs[0] + vs[1] + vs[2] + vs[3]
                q = vs[0] * vs[0] + vs[1] * vs[1] + vs[2] * vs[2] + vs[3] * vs[3]
                m = jnp.sum(s) * (1.0 / D)
                var = jnp.sum(q) * (1.0 / D) - m * m
                sc = _rsqrt(var + EPS)
                for j in range(D // LANES):
                    wj, bj = wb[j]
                    rows_v[r, pl.ds(j * LANES, LANES)] = ((vs[j] - m) * sc) * wj + bj
            pltpu.sync_copy(rows_v, out_hbm.at[pl.ds(off, CHUNK)])
            return carry

        lax.fori_loop(0, n_chunks, chunk_body, 0)

    return body


def kernel(x, table, ln_weight, ln_bias):
    batch, hist = x.shape
    n_rows = batch * hist
    xt = jnp.swapaxes(x, 0, 1).reshape(n_rows)
    out = _make_sc_call(n_rows)(xt, table, ln_weight, ln_bias)
    return out.reshape(hist, batch, D)


# speedup vs baseline: 2.0587x; 1.6303x over previous
import functools

import jax
import jax.numpy as jnp
from jax import lax
from jax.experimental import pallas as pl
from jax.experimental.pallas import tpu as pltpu
from jax.experimental.pallas import tpu_sc as plsc

D = 64
LANES = 16
CHUNK = 128
NBUF = 4
EPS = 1e-5


def _rsqrt(x):
    i = lax.bitcast_convert_type(x, jnp.int32)
    i = jnp.int32(0x5F3759DF) - (i >> 1)
    y = lax.bitcast_convert_type(i, jnp.float32)
    for _ in range(3):
        y = y * (1.5 - (0.5 * x) * y * y)
    return y


@functools.cache
def _make_sc_call(n_rows):
    info = plsc.get_sparse_core_info()
    num_cores = info.num_cores
    nw = num_cores * info.num_subcores
    per_w = n_rows // nw
    n_chunks = per_w // CHUNK
    assert per_w * nw == n_rows and n_chunks * CHUNK == per_w
    assert n_chunks % NBUF == 0 and n_chunks >= 2 * NBUF
    mesh = plsc.VectorSubcoreMesh(core_axis_name="c", subcore_axis_name="s")

    @functools.partial(
        pl.kernel,
        mesh=mesh,
        compiler_params=pltpu.CompilerParams(
            needs_layout_passes=False, use_tc_tiling_on_sc=False),
        out_type=jax.ShapeDtypeStruct((n_rows, D), jnp.float32),
        scratch_types=(
            [pltpu.VMEM((per_w,), jnp.int32)]
            + [pltpu.VMEM((CHUNK, D), jnp.float32) for _ in range(NBUF)]
            + [pltpu.VMEM((D,), jnp.float32), pltpu.VMEM((D,), jnp.float32)]
            + [pltpu.SemaphoreType.DMA for _ in range(2 * NBUF)]
        ),
    )
    def body(idx_hbm, table_hbm, w_hbm, b_hbm, out_hbm,
             idx_v, rb0, rb1, rb2, rb3, w_v, b_v,
             g0, g1, g2, g3, o0, o1, o2, o3):
        rows = [rb0, rb1, rb2, rb3]
        gsem = [g0, g1, g2, g3]
        osem = [o0, o1, o2, o3]
        wid = lax.axis_index("s") * num_cores + lax.axis_index("c")
        base = wid * per_w
        pltpu.sync_copy(w_hbm, w_v)
        pltpu.sync_copy(b_hbm, b_v)
        pltpu.sync_copy(idx_hbm.at[pl.ds(base, per_w)], idx_v)

        def gather(ci, b):
            off = pl.multiple_of(ci * CHUNK, CHUNK)
            return pltpu.make_async_copy(
                table_hbm.at[idx_v.at[pl.ds(off, CHUNK)]], rows[b], gsem[b])

        def out_copy(ci, b):
            off = pl.multiple_of(base + ci * CHUNK, CHUNK)
            return pltpu.make_async_copy(
                rows[b], out_hbm.at[pl.ds(off, CHUNK)], osem[b])

        gather(0, 0).start()
        gather(1, 1).start()

        def ln_chunk(rv):
            wb = [(w_v[pl.ds(j * LANES, LANES)],
                   b_v[pl.ds(j * LANES, LANES)])
                  for j in range(D // LANES)]

            def group(g, carry):
                gv = rv.at[pl.ds(g * LANES, LANES)]
                for r in range(LANES):
                    vs = [gv[r, pl.ds(j * LANES, LANES)]
                          for j in range(D // LANES)]
                    s = vs[0] + vs[1] + vs[2] + vs[3]
                    q = (vs[0] * vs[0] + vs[1] * vs[1]
                         + vs[2] * vs[2] + vs[3] * vs[3])
                    m = jnp.sum(s) * (1.0 / D)
                    var = jnp.sum(q) * (1.0 / D) - m * m
                    sc = _rsqrt(var + EPS)
                    for j in range(D // LANES):
                        wj, bj = wb[j]
                        gv[r, pl.ds(j * LANES, LANES)] = (
                            ((vs[j] - m) * sc) * wj + bj)
                return carry

            lax.fori_loop(0, CHUNK // LANES, group, 0)

        def outer(oc, carry):
            for b in range(NBUF):
                ci = oc * NBUF + b
                gather(ci, b).wait()
                ln_chunk(rows[b])
                out_copy(ci, b).start()
                bg = (b + 2) % NBUF

                @pl.when(ci + 2 < n_chunks)
                def _issue():
                    @pl.when(ci >= 2)
                    def _drain():
                        out_copy(ci - 2, bg).wait()
                    gather(ci + 2, bg).start()
            return carry

        lax.fori_loop(0, n_chunks // NBUF, outer, 0)
        for b in range(NBUF):
            out_copy(n_chunks - NBUF + b, b).wait()

    return body


def kernel(x, table, ln_weight, ln_bias):
    batch, hist = x.shape
    n_rows = batch * hist
    xt = jnp.swapaxes(x, 0, 1).reshape(n_rows)
    out = _make_sc_call(n_rows)(xt, table, ln_weight, ln_bias)
    return out.reshape(hist, batch, D)
